# Initial kernel scaffold; baseline (speedup 1.0000x reference)
import jax
import jax.numpy as jnp
from jax.experimental import pallas as pl


def kernel(aa, pos, is_cdr3, edge_index, batch, emb_table, W1, b1, Ws1, bs1, W2, b2, Ws2, bs2, W3, b3, Ws3, bs3, Wm1, bm1, Wm2, bm2, Wm3, bm3):
    raise NotImplementedError("write your pallas kernel here")


# speedup vs baseline: 1.1204x; 1.1204x over previous
import functools

import jax
import jax.numpy as jnp
from jax.experimental import pallas as pl

G = 512
RATIO = 0.5


def _topk_keep(score, batch, valid):
    n = score.shape[0]
    counts = jnp.zeros((G,), jnp.int32).at[batch].add(valid.astype(jnp.int32))
    k = jnp.ceil(RATIO * counts.astype(jnp.float32)).astype(jnp.int32)
    b = jax.lax.bitcast_convert_type(score, jnp.int32)
    u = jnp.where(b < 0, ~b, b ^ jnp.int32(-2147483648)).astype(jnp.uint32)
    u = jnp.where(valid, u, jnp.uint32(0))

    def body(i, lohi):
        lo, hi = lohi
        mid = lo + (hi - lo) // 2 + ((hi - lo) % 2)
        cnt = jnp.zeros((G,), jnp.int32).at[batch].add(
            (valid & (u >= mid[batch])).astype(jnp.int32))
        ge = cnt >= k
        return jnp.where(ge, mid, lo), jnp.where(ge, hi, mid - 1)

    lo, hi = jax.lax.fori_loop(
        0, 32, body,
        (jnp.zeros((G,), jnp.uint32), jnp.full((G,), jnp.uint32(0xFFFFFFFF))))
    t = lo[batch]
    gt = valid & (u > t)
    eq = valid & (u == t)
    cnt_gt = jnp.zeros((G,), jnp.int32).at[batch].add(gt.astype(jnp.int32))
    seg_tot = jnp.zeros((G,), jnp.int32).at[batch].add(eq.astype(jnp.int32))
    seg_start = jnp.concatenate([jnp.zeros((1,), jnp.int32), jnp.cumsum(seg_tot)[:-1]])
    tie_rank = jnp.cumsum(eq.astype(jnp.int32)) - eq.astype(jnp.int32) - seg_start[batch]
    keep = gt | (eq & (cnt_gt[batch] + tie_rank < k[batch]))
    return keep & (counts[batch] > 0)


def _mlp_body(g_ref, w1_ref, b1_ref, w2_ref, b2_ref, w3_ref, b3_ref, out_ref):
    h = jnp.maximum(g_ref[...] @ w1_ref[...] + b1_ref[...], 0.0)
    h = jnp.maximum(h @ w2_ref[...] + b2_ref[...], 0.0)
    out_ref[...] = h @ w3_ref[...] + b3_ref[...]


def _mlp_head(g, Wm1, bm1, Wm2, bm2, Wm3, bm3):
    out = pl.pallas_call(
        _mlp_body,
        out_shape=jax.ShapeDtypeStruct((G, 1), jnp.float32),
    )(g, Wm1, bm1[None, :], Wm2, bm2[None, :], Wm3, bm3[None, :])
    return out.reshape(-1)


def kernel(aa, pos, is_cdr3, edge_index, batch, emb_table, W1, b1, Ws1, bs1, W2, b2,
           Ws2, bs2, W3, b3, Ws3, bs3, Wm1, bm1, Wm2, bm2, Wm3, bm3):
    n = aa.shape[0]
    src, dst = edge_index[0], edge_index[1]
    x = jnp.concatenate([emb_table[aa], pos, is_cdr3], axis=1)
    valid = jnp.ones((n,), bool)
    ew = jnp.ones((src.shape[0],), jnp.float32)
    outs = []
    for (W, b, Ws, bs) in ((W1, b1, Ws1, bs1), (W2, b2, Ws2, bs2), (W3, b3, Ws3, bs3)):
        deg = jnp.ones((n,), jnp.float32).at[dst].add(ew)
        dinv = 1.0 / jnp.sqrt(deg)
        coef = dinv[src] * dinv[dst] * ew
        xw = x @ W
        agg = (xw * (dinv * dinv)[:, None]).at[dst].add(xw[src] * coef[:, None])
        h = jax.nn.relu(agg + b)
        hw = (h @ Ws).reshape(-1)
        score = hw * dinv * dinv + jnp.zeros((n,), jnp.float32).at[dst].add(hw[src] * coef) + bs[0]
        keep = _topk_keep(score, batch, valid)
        x = jnp.where(keep[:, None], h * jnp.tanh(score)[:, None], 0.0)
        ew = ew * (keep[src] & keep[dst]).astype(jnp.float32)
        valid = keep
        cnt = jnp.zeros((G,), jnp.int32).at[batch].add(keep.astype(jnp.int32)).astype(jnp.float32)
        gmax = jax.ops.segment_max(jnp.where(keep[:, None], x, -jnp.inf), batch, num_segments=G)
        gmax = jnp.where(cnt[:, None] > 0, gmax, 0.0)
        gmean = jax.ops.segment_sum(x, batch, num_segments=G) / jnp.maximum(cnt, 1.0)[:, None]
        outs.append(jnp.concatenate([gmax, gmean], axis=1))
    g = outs[0] + outs[1] + outs[2]
    return _mlp_head(g, Wm1, bm1, Wm2, bm2, Wm3, bm3)


# speedup vs baseline: 1.8092x; 1.6148x over previous
import functools

import jax
import jax.numpy as jnp
from jax import lax
from jax.experimental import pallas as pl
from jax.experimental.pallas import tpu as pltpu
from jax.experimental.pallas import tpu_sc as plsc

G = 512
RATIO = 0.5
N = 100000
E = 1600000
NS = 16
CHUNK = 80
ZROWS = 1000
NZCH = N // ZROWS


def _feat_conv_body(y2_hbm, src_hbm, dst_hbm, out_hbm, acc_sh, zero_v, src_v,
                    dst_v, idx_v, rows_v, sem):
    c = lax.axis_index("c")
    s = lax.axis_index("s")
    zero_v[...] = jnp.zeros_like(zero_v)
    for j in range(-(-NZCH // NS)):
        cid = s + NS * j
        @pl.when(cid < NZCH)
        def _():
            pltpu.sync_copy(zero_v, acc_sh.at[pl.ds(cid * ZROWS, ZROWS)])
    plsc.subcore_barrier()

    nsteps = (E // NS) // CHUNK
    base0 = s * (E // NS)

    def step(i, carry):
        base = base0 + i * CHUNK
        pltpu.sync_copy(src_hbm.at[pl.ds(base, CHUNK)], src_v)
        pltpu.sync_copy(dst_hbm.at[pl.ds(base, CHUNK)], dst_v)
        for j in range(CHUNK // 16):
            idx_v[pl.ds(j * 16, 16)] = src_v[pl.ds(j * 16, 16)] * 2 + c
        pltpu.async_copy(y2_hbm.at[idx_v], rows_v, sem).wait()
        pltpu.sync_copy(rows_v, acc_sh.at[dst_v], add=True)
        return carry

    lax.fori_loop(0, nsteps, step, 0)
    plsc.subcore_barrier()
    for j in range(-(-NZCH // NS)):
        cid = s + NS * j
        @pl.when(cid < NZCH)
        def _():
            r0 = cid * ZROWS
            pltpu.sync_copy(acc_sh.at[pl.ds(r0, ZROWS)], out_hbm.at[c, pl.ds(r0, ZROWS)])


@jax.jit
def _feat_conv(y, src, dst):
    y2 = y.reshape(2 * N, 16)
    mesh = plsc.VectorSubcoreMesh(core_axis_name="c", subcore_axis_name="s")
    out = pl.kernel(
        _feat_conv_body,
        out_type=jax.ShapeDtypeStruct((2, N, 16), jnp.float32),
        mesh=mesh,
        compiler_params=pltpu.CompilerParams(use_tc_tiling_on_sc=False),
        scratch_types=[
            pltpu.VMEM_SHARED((N, 16), jnp.float32),
            pltpu.VMEM((ZROWS, 16), jnp.float32),
            pltpu.VMEM((CHUNK,), jnp.int32),
            pltpu.VMEM((CHUNK,), jnp.int32),
            pltpu.VMEM((CHUNK,), jnp.int32),
            pltpu.VMEM((CHUNK, 16), jnp.float32),
            pltpu.SemaphoreType.DMA,
        ],
    )(y2, src, dst)
    return jnp.concatenate([out[0], out[1]], axis=1)


def _topk_keep(score, batch, valid):
    counts = jnp.zeros((G,), jnp.int32).at[batch].add(valid.astype(jnp.int32))
    k = jnp.ceil(RATIO * counts.astype(jnp.float32)).astype(jnp.int32)
    b = lax.bitcast_convert_type(score, jnp.int32)
    u = jnp.where(b < 0, ~b, b ^ jnp.int32(-2147483648)).astype(jnp.uint32)
    u = jnp.where(valid, u, jnp.uint32(0))

    def body(i, lohi):
        lo, hi = lohi
        mid = lo + (hi - lo) // 2 + ((hi - lo) % 2)
        cnt = jnp.zeros((G,), jnp.int32).at[batch].add(
            (valid & (u >= mid[batch])).astype(jnp.int32))
        ge = cnt >= k
        return jnp.where(ge, mid, lo), jnp.where(ge, hi, mid - 1)

    lo, hi = lax.fori_loop(
        0, 32, body,
        (jnp.zeros((G,), jnp.uint32), jnp.full((G,), jnp.uint32(0xFFFFFFFF))))
    t = lo[batch]
    gt = valid & (u > t)
    eq = valid & (u == t)
    cnt_gt = jnp.zeros((G,), jnp.int32).at[batch].add(gt.astype(jnp.int32))
    seg_tot = jnp.zeros((G,), jnp.int32).at[batch].add(eq.astype(jnp.int32))
    seg_start = jnp.concatenate([jnp.zeros((1,), jnp.int32), jnp.cumsum(seg_tot)[:-1]])
    tie_rank = jnp.cumsum(eq.astype(jnp.int32)) - eq.astype(jnp.int32) - seg_start[batch]
    keep = gt | (eq & (cnt_gt[batch] + tie_rank < k[batch]))
    return keep & (counts[batch] > 0)


def _mlp_body(g_ref, w1_ref, b1_ref, w2_ref, b2_ref, w3_ref, b3_ref, out_ref):
    h = jnp.maximum(g_ref[...] @ w1_ref[...] + b1_ref[...], 0.0)
    h = jnp.maximum(h @ w2_ref[...] + b2_ref[...], 0.0)
    out_ref[...] = h @ w3_ref[...] + b3_ref[...]


def _mlp_head(g, Wm1, bm1, Wm2, bm2, Wm3, bm3):
    out = pl.pallas_call(
        _mlp_body,
        out_shape=jax.ShapeDtypeStruct((G, 1), jnp.float32),
    )(g, Wm1, bm1[None, :], Wm2, bm2[None, :], Wm3, bm3[None, :])
    return out.reshape(-1)


def kernel(aa, pos, is_cdr3, edge_index, batch, emb_table, W1, b1, Ws1, bs1, W2, b2,
           Ws2, bs2, W3, b3, Ws3, bs3, Wm1, bm1, Wm2, bm2, Wm3, bm3):
    n = aa.shape[0]
    src, dst = edge_index[0], edge_index[1]
    x = jnp.concatenate([emb_table[aa], pos, is_cdr3], axis=1)
    valid = jnp.ones((n,), bool)
    keepf = jnp.ones((n,), jnp.float32)
    outs = []
    for (W, b, Ws, bs) in ((W1, b1, Ws1, bs1), (W2, b2, Ws2, bs2), (W3, b3, Ws3, bs3)):
        deg = jnp.ones((n,), jnp.float32).at[dst].add(keepf[src])
        dinv = lax.rsqrt(deg)
        xw = x @ W
        y = xw * dinv[:, None]
        agg = dinv[:, None] * (_feat_conv(y, src, dst) + y)
        h = jax.nn.relu(agg + b)
        hw = (h @ Ws).reshape(-1)
        z = jnp.where(valid, hw * dinv, 0.0)
        sagg = jnp.zeros((n,), jnp.float32).at[dst].add(z[src])
        score = dinv * (sagg + z) + bs[0]
        keep = _topk_keep(score, batch, valid)
        x = jnp.where(keep[:, None], h * jnp.tanh(score)[:, None], 0.0)
        valid = keep
        keepf = keep.astype(jnp.float32)
        cnt = jnp.zeros((G,), jnp.int32).at[batch].add(keep.astype(jnp.int32)).astype(jnp.float32)
        gmax = jax.ops.segment_max(jnp.where(keep[:, None], x, -jnp.inf), batch, num_segments=G)
        gmax = jnp.where(cnt[:, None] > 0, gmax, 0.0)
        gmean = jax.ops.segment_sum(x, batch, num_segments=G) / jnp.maximum(cnt, 1.0)[:, None]
        outs.append(jnp.concatenate([gmax, gmean], axis=1))
    g = outs[0] + outs[1] + outs[2]
    return _mlp_head(g, Wm1, bm1, Wm2, bm2, Wm3, bm3)


# speedup vs baseline: 2.6255x; 1.4511x over previous
import functools

import jax
import jax.numpy as jnp
from jax import lax
from jax.experimental import pallas as pl
from jax.experimental.pallas import tpu as pltpu
from jax.experimental.pallas import tpu_sc as plsc

G = 512
RATIO = 0.5
N = 100000
E = 1600000
NS = 16
CHUNK = 80
ZROWS = 1000
NZCH = N // ZROWS


def _feat_conv_body(y2_hbm, src_hbm, dst_hbm, out_hbm, acc_sh, zero_v, src_v,
                    dst_v, idx_v, rows_v, sem):
    c = lax.axis_index("c")
    s = lax.axis_index("s")
    zero_v[...] = jnp.zeros_like(zero_v)
    for j in range(-(-NZCH // NS)):
        cid = s + NS * j
        @pl.when(cid < NZCH)
        def _():
            pltpu.sync_copy(zero_v, acc_sh.at[pl.ds(cid * ZROWS, ZROWS)])
    plsc.subcore_barrier()

    nsteps = (E // NS) // CHUNK
    base0 = s * (E // NS)

    def step(i, carry):
        base = base0 + i * CHUNK
        pltpu.sync_copy(src_hbm.at[pl.ds(base, CHUNK)], src_v)
        pltpu.sync_copy(dst_hbm.at[pl.ds(base, CHUNK)], dst_v)
        for j in range(CHUNK // 16):
            idx_v[pl.ds(j * 16, 16)] = src_v[pl.ds(j * 16, 16)] * 2 + c
        pltpu.async_copy(y2_hbm.at[idx_v], rows_v, sem).wait()
        pltpu.sync_copy(rows_v, acc_sh.at[dst_v], add=True)
        return carry

    lax.fori_loop(0, nsteps, step, 0)
    plsc.subcore_barrier()
    for j in range(-(-NZCH // NS)):
        cid = s + NS * j
        @pl.when(cid < NZCH)
        def _():
            r0 = cid * ZROWS
            pltpu.sync_copy(acc_sh.at[pl.ds(r0, ZROWS)], out_hbm.at[c, pl.ds(r0, ZROWS)])


@jax.jit
def _feat_conv(y, src, dst):
    y2 = y.reshape(2 * N, 16)
    mesh = plsc.VectorSubcoreMesh(core_axis_name="c", subcore_axis_name="s")
    out = pl.kernel(
        _feat_conv_body,
        out_type=jax.ShapeDtypeStruct((2, N, 16), jnp.float32),
        mesh=mesh,
        compiler_params=pltpu.CompilerParams(use_tc_tiling_on_sc=False),
        scratch_types=[
            pltpu.VMEM_SHARED((N, 16), jnp.float32),
            pltpu.VMEM((ZROWS, 16), jnp.float32),
            pltpu.VMEM((CHUNK,), jnp.int32),
            pltpu.VMEM((CHUNK,), jnp.int32),
            pltpu.VMEM((CHUNK,), jnp.int32),
            pltpu.VMEM((CHUNK, 16), jnp.float32),
            pltpu.SemaphoreType.DMA,
        ],
    )(y2, src, dst)
    return jnp.concatenate([out[0], out[1]], axis=1)


CL = 2048
NPAD = N + CL


def _topk_body(u_hbm, batch_hbm, starts_hbm, k_hbm, out_hbm, ubuf, bbuf, sbuf,
               kbuf, mid_buf, t_buf, m_buf, cnt_buf, idx_buf, val_buf, sem):
    c = lax.axis_index("c")
    s = lax.axis_index("s")
    w = s * 2 + c
    g0 = w * 16
    pltpu.sync_copy(starts_hbm.at[pl.ds(g0, 24)], sbuf)
    pltpu.sync_copy(k_hbm.at[pl.ds(g0, 16)], kbuf)
    a = sbuf[pl.ds(0, 16)][0]
    end = sbuf[pl.ds(8, 16)][8]
    base0 = (a // 8) * 8
    nch = (end - base0 + (CL - 1)) // CL
    k_v = kbuf[...]
    dump_hi = jnp.full((16,), -1, jnp.int32)
    dump_lo = jnp.zeros((16,), jnp.int32)
    iota = lax.iota(jnp.int32, 16)

    def _slot(bv):
        rel = bv - g0
        return jnp.minimum(rel.astype(jnp.uint32), jnp.uint32(16)).astype(jnp.int32)

    def _count_pass(ind_fn, nslots):
        for q in range(nslots // 16):
            cnt_buf[pl.ds(q * 16, 16)] = jnp.zeros((16,), jnp.int32)

        def chunk(ch, carry):
            b0 = base0 + ch * CL
            pltpu.sync_copy(u_hbm.at[pl.ds(b0, CL)], ubuf)
            pltpu.sync_copy(batch_hbm.at[pl.ds(b0, CL)], bbuf)

            def step(j, carry2):
                uv = ubuf[pl.ds(j * 16, 16)].astype(jnp.uint32)
                bv = bbuf[pl.ds(j * 16, 16)]
                slot = _slot(bv)
                idxv = b0 + j * 16 + iota
                ind_fn(uv, slot, idxv)
                return carry2

            lax.fori_loop(0, CL // 16, step, 0)
            return carry

        lax.fori_loop(0, nch, chunk, 0)

    def vsearch(i, lohi):
        lo, hi = lohi
        mid = lo + (hi - lo) // 2 + (hi - lo) % 2
        mid_buf[pl.ds(0, 16)] = mid.astype(jnp.int32)
        mid_buf[pl.ds(16, 16)] = dump_hi

        def ind(uv, slot, idxv):
            mval = plsc.load_gather(mid_buf, [slot]).astype(jnp.uint32)
            plsc.addupdate_scatter(cnt_buf, [slot],
                                   jnp.where(uv >= mval, 1, 0).astype(jnp.int32))

        _count_pass(ind, 32)
        cnt = cnt_buf[pl.ds(0, 16)]
        ge = cnt >= k_v
        return (jnp.where(ge, mid, lo), jnp.where(ge, hi, mid - jnp.uint32(1)))

    lo, hi = lax.fori_loop(0, 32, vsearch, (jnp.zeros((16,), jnp.uint32),
                                            jnp.full((16,), 0xFFFFFFFF, jnp.uint32)))
    t = lo
    t_buf[pl.ds(0, 16)] = t.astype(jnp.int32)
    t_buf[pl.ds(16, 16)] = dump_hi

    def ind_gt_eq(uv, slot, idxv):
        tval = plsc.load_gather(t_buf, [slot]).astype(jnp.uint32)
        plsc.addupdate_scatter(cnt_buf, [slot],
                               jnp.where(uv > tval, 1, 0).astype(jnp.int32))
        plsc.addupdate_scatter(cnt_buf, [slot + 24],
                               jnp.where(uv == tval, 1, 0).astype(jnp.int32))

    _count_pass(ind_gt_eq, 48)
    n_take = k_v - cnt_buf[pl.ds(0, 16)]

    def isearch(i, lohi2):
        lo2, hi2 = lohi2
        mid = (lo2 + hi2) // 2
        m_buf[pl.ds(0, 16)] = mid
        m_buf[pl.ds(16, 16)] = dump_lo

        def ind(uv, slot, idxv):
            tval = plsc.load_gather(t_buf, [slot]).astype(jnp.uint32)
            mval = plsc.load_gather(m_buf, [slot])
            eq = (uv == tval) & (idxv < mval)
            plsc.addupdate_scatter(cnt_buf, [slot],
                                   jnp.where(eq, 1, 0).astype(jnp.int32))

        _count_pass(ind, 32)
        cnt = cnt_buf[pl.ds(0, 16)]
        ge = cnt >= n_take
        return (jnp.where(ge, lo2, mid + 1), jnp.where(ge, mid, hi2))

    lo2, hi2 = lax.fori_loop(0, 17, isearch, (jnp.zeros((16,), jnp.int32),
                                              jnp.full((16,), N, jnp.int32)))
    m_buf[pl.ds(0, 16)] = lo2
    m_buf[pl.ds(16, 16)] = dump_lo

    def wchunk(ch, carry):
        b0 = base0 + ch * CL
        pltpu.sync_copy(u_hbm.at[pl.ds(b0, CL)], ubuf)
        pltpu.sync_copy(batch_hbm.at[pl.ds(b0, CL)], bbuf)

        def sub(q, carry2):
            for j in range(8):
                uv = ubuf[pl.ds(q * 128 + j * 16, 16)].astype(jnp.uint32)
                bv = bbuf[pl.ds(q * 128 + j * 16, 16)]
                slot = _slot(bv)
                idxv = b0 + q * 128 + j * 16 + iota
                tval = plsc.load_gather(t_buf, [slot]).astype(jnp.uint32)
                mval = plsc.load_gather(m_buf, [slot])
                kp = (uv > tval) | ((uv == tval) & (idxv < mval))
                val_buf[pl.ds(j * 16, 16)] = jnp.where(kp, 1.0, 0.0)
                idx_buf[pl.ds(j * 16, 16)] = jnp.where(slot < 16, idxv, N)
            pltpu.async_copy(val_buf, out_hbm.at[idx_buf], sem).wait()
            return carry2

        lax.fori_loop(0, CL // 128, sub, 0)
        return carry

    lax.fori_loop(0, nch, wchunk, 0)


@jax.jit
def _topk_sc(u, batch_pad, starts, k):
    mesh = plsc.VectorSubcoreMesh(core_axis_name="c", subcore_axis_name="s")
    out = pl.kernel(
        _topk_body,
        out_type=jax.ShapeDtypeStruct((N + 8,), jnp.float32),
        mesh=mesh,
        compiler_params=pltpu.CompilerParams(use_tc_tiling_on_sc=False,
                                             needs_layout_passes=False),
        scratch_types=[
            pltpu.VMEM((CL,), jnp.int32),
            pltpu.VMEM((CL,), jnp.int32),
            pltpu.VMEM((24,), jnp.int32),
            pltpu.VMEM((16,), jnp.int32),
            pltpu.VMEM((32,), jnp.int32),
            pltpu.VMEM((32,), jnp.int32),
            pltpu.VMEM((32,), jnp.int32),
            pltpu.VMEM((48,), jnp.int32),
            pltpu.VMEM((128,), jnp.int32),
            pltpu.VMEM((128,), jnp.float32),
            pltpu.SemaphoreType.DMA,
        ],
    )(u, batch_pad, starts, k)
    return out[:N]


def _mlp_body(g_ref, w1_ref, b1_ref, w2_ref, b2_ref, w3_ref, b3_ref, out_ref):
    h = jnp.maximum(g_ref[...] @ w1_ref[...] + b1_ref[...], 0.0)
    h = jnp.maximum(h @ w2_ref[...] + b2_ref[...], 0.0)
    out_ref[...] = h @ w3_ref[...] + b3_ref[...]


def _mlp_head(g, Wm1, bm1, Wm2, bm2, Wm3, bm3):
    out = pl.pallas_call(
        _mlp_body,
        out_shape=jax.ShapeDtypeStruct((G, 1), jnp.float32),
    )(g, Wm1, bm1[None, :], Wm2, bm2[None, :], Wm3, bm3[None, :])
    return out.reshape(-1)


def kernel(aa, pos, is_cdr3, edge_index, batch, emb_table, W1, b1, Ws1, bs1, W2, b2,
           Ws2, bs2, W3, b3, Ws3, bs3, Wm1, bm1, Wm2, bm2, Wm3, bm3):
    n = aa.shape[0]
    src, dst = edge_index[0], edge_index[1]
    x = jnp.concatenate([emb_table[aa], pos, is_cdr3], axis=1)
    valid = jnp.ones((n,), bool)
    keepf = jnp.ones((n,), jnp.float32)
    batch = batch.astype(jnp.int32)
    starts = jnp.searchsorted(batch, jnp.arange(G + 1, dtype=jnp.int32)).astype(jnp.int32)
    starts_pad = jnp.concatenate([starts, jnp.full((7,), N, jnp.int32)])
    batch_pad = jnp.concatenate([batch, jnp.full((CL,), G, jnp.int32)])
    counts = starts[1:] - starts[:-1]
    outs = []
    for (W, b, Ws, bs) in ((W1, b1, Ws1, bs1), (W2, b2, Ws2, bs2), (W3, b3, Ws3, bs3)):
        deg = jnp.ones((n,), jnp.float32).at[dst].add(keepf[src])
        dinv = lax.rsqrt(deg)
        xw = x @ W
        y = xw * dinv[:, None]
        agg = dinv[:, None] * (_feat_conv(y, src, dst) + y)
        h = jax.nn.relu(agg + b)
        hw = (h @ Ws).reshape(-1)
        z = jnp.where(valid, hw * dinv, 0.0)
        sagg = jnp.zeros((n,), jnp.float32).at[dst].add(z[src])
        score = dinv * (sagg + z) + bs[0]
        bbits = lax.bitcast_convert_type(score, jnp.int32)
        u = jnp.where(bbits < 0, ~bbits, bbits ^ jnp.int32(-2147483648)).astype(jnp.uint32)
        u = jnp.where(valid, u, jnp.uint32(0))
        u_pad = jnp.concatenate([lax.bitcast_convert_type(u, jnp.int32),
                                 jnp.zeros((CL,), jnp.int32)])
        k = (counts + 1) // 2
        keepf = _topk_sc(u_pad, batch_pad, starts_pad, k)
        keep = keepf > 0.5
        counts = k
        x = jnp.where(keep[:, None], h * jnp.tanh(score)[:, None], 0.0)
        valid = keep
        cnt = k.astype(jnp.float32)
        gmax = jax.ops.segment_max(jnp.where(keep[:, None], x, -jnp.inf), batch, num_segments=G)
        gmax = jnp.where(cnt[:, None] > 0, gmax, 0.0)
        gmean = jax.ops.segment_sum(x, batch, num_segments=G) / jnp.maximum(cnt, 1.0)[:, None]
        outs.append(jnp.concatenate([gmax, gmean], axis=1))
    g = outs[0] + outs[1] + outs[2]
    return _mlp_head(g, Wm1, bm1, Wm2, bm2, Wm3, bm3)
